# baseline (speedup 1.0000x reference)
import functools

import jax
import jax.numpy as jnp
from jax import lax
from jax.experimental import pallas as pl
from jax.experimental.pallas import tpu as pltpu
from jax.experimental.pallas import tpu_sc as plsc

D = 768
E = 8
K = 2
BT = 2048

NC = 2
NS = 16
NW = NC * NS
LANES = 16


def _gate_body(x_ref, wt_ref, out_ref):
    out_ref[...] = jnp.dot(x_ref[...], wt_ref[...],
                           preferred_element_type=jnp.float32)


def _gate_logits(x_flat, wt):
    n_tok = x_flat.shape[0]
    grid = (n_tok // BT,)
    return pl.pallas_call(
        _gate_body,
        grid=grid,
        in_specs=[
            pl.BlockSpec((BT, D), lambda i: (i, 0)),
            pl.BlockSpec((D, E), lambda i: (0, 0)),
        ],
        out_specs=pl.BlockSpec((BT, E), lambda i: (i, 0)),
        out_shape=jax.ShapeDtypeStruct((n_tok, E), jnp.float32),
    )(x_flat, wt)


def _make_router(n_tok):
    tpw = n_tok // NW
    steps = tpw // LANES

    mesh = plsc.VectorSubcoreMesh(core_axis_name="c", subcore_axis_name="s")

    @functools.partial(
        pl.kernel,
        mesh=mesh,
        compiler_params=pltpu.CompilerParams(needs_layout_passes=False),
        out_type=[
            jax.ShapeDtypeStruct((n_tok * K,), jnp.float32),
            jax.ShapeDtypeStruct((n_tok * K,), jnp.int32),
        ],
        scratch_types=[
            pltpu.VMEM((tpw * E,), jnp.float32),
            pltpu.VMEM((tpw * K,), jnp.float32),
            pltpu.VMEM((tpw * K,), jnp.int32),
        ],
    )
    def router(logits_hbm, gates_hbm, idx_hbm, lv, gv, iv):
        wid = lax.axis_index("s") * NC + lax.axis_index("c")
        base = wid * tpw
        pltpu.sync_copy(logits_hbm.at[pl.ds(base * E, tpw * E)], lv)

        lane = lax.iota(jnp.int32, LANES)

        def step(t, carry):
            tok0 = t * LANES
            src = (tok0 + lane) * E
            ls = [plsc.load_gather(lv, [src + e]) for e in range(E)]

            m1 = ls[0]
            i1 = jnp.zeros((LANES,), jnp.int32)
            for e in range(1, E):
                gt = ls[e] > m1
                m1 = jnp.where(gt, ls[e], m1)
                i1 = jnp.where(gt, e, i1)

            neg = jnp.float32(-1e30)
            m2 = jnp.full((LANES,), neg, jnp.float32)
            i2 = jnp.zeros((LANES,), jnp.int32)
            for e in range(E):
                cand = jnp.where(i1 == e, neg, ls[e])
                gt = cand > m2
                m2 = jnp.where(gt, cand, m2)
                i2 = jnp.where(gt, e, i2)

            t2 = jnp.exp(m2 - m1)
            g1 = 1.0 / (1.0 + t2)
            g2 = t2 * g1

            dst = (tok0 + lane) * K
            plsc.store_scatter(gv, [dst], g1)
            plsc.store_scatter(gv, [dst + 1], g2)
            plsc.store_scatter(iv, [dst], i1)
            plsc.store_scatter(iv, [dst + 1], i2)
            return carry

        lax.fori_loop(0, steps, step, 0)

        pltpu.sync_copy(gv, gates_hbm.at[pl.ds(base * K, tpw * K)])
        pltpu.sync_copy(iv, idx_hbm.at[pl.ds(base * K, tpw * K)])

    return router


def kernel(x, W):
    B, S, _ = x.shape
    n_tok = B * S
    x_flat = x.reshape(n_tok, D)
    logits = _gate_logits(x_flat, W.T)
    gates_flat, idx_flat = _make_router(n_tok)(logits.reshape(-1))
    return (
        gates_flat.reshape(B, S, K),
        idx_flat.reshape(B, S, K),
        logits.reshape(B, S, E),
    )


# speedup vs baseline: 2.3218x; 2.3218x over previous
import functools

import jax
import jax.numpy as jnp
from jax import lax
from jax.experimental import pallas as pl
from jax.experimental.pallas import tpu as pltpu
from jax.experimental.pallas import tpu_sc as plsc

D = 768
E = 8
K = 2
BT = 2048

NC = 2
NS = 16
NW = NC * NS
LANES = 16


def _gate_body(x_ref, wt_ref, outt_ref):
    acc = jnp.dot(x_ref[...], wt_ref[...], preferred_element_type=jnp.float32)
    outt_ref[...] = jnp.transpose(acc.reshape(BT // 128, 128, E), (0, 2, 1))


def _gate_logits_t(x_flat, wt):
    n_tok = x_flat.shape[0]
    grid = (n_tok // BT,)
    return pl.pallas_call(
        _gate_body,
        grid=grid,
        in_specs=[
            pl.BlockSpec((BT, D), lambda i: (i, 0)),
            pl.BlockSpec((D, E), lambda i: (0, 0)),
        ],
        out_specs=pl.BlockSpec((BT // 128, E, 128), lambda i: (i, 0, 0)),
        out_shape=jax.ShapeDtypeStruct((n_tok // 128, E, 128), jnp.float32),
    )(x_flat, wt)


def _make_router(n_tiles):
    tpw = n_tiles // NW

    mesh = plsc.VectorSubcoreMesh(core_axis_name="c", subcore_axis_name="s")

    @functools.partial(
        pl.kernel,
        mesh=mesh,
        compiler_params=pltpu.CompilerParams(needs_layout_passes=False),
        out_type=[
            jax.ShapeDtypeStruct((n_tiles, K, 128), jnp.float32),
            jax.ShapeDtypeStruct((n_tiles, K, 128), jnp.int32),
        ],
        scratch_types=[
            pltpu.VMEM((tpw, E, 128), jnp.float32),
            pltpu.VMEM((tpw, K, 128), jnp.float32),
            pltpu.VMEM((tpw, K, 128), jnp.int32),
        ],
    )
    def router(lt_hbm, gates_hbm, idx_hbm, lv, gv, iv):
        wid = lax.axis_index("s") * NC + lax.axis_index("c")
        base = wid * tpw
        pltpu.sync_copy(lt_hbm.at[pl.ds(base, tpw)], lv)

        zero = jnp.zeros((LANES,), jnp.int32)
        neg = jnp.float32(-1e30)
        negv = jnp.full((LANES,), neg, jnp.float32)

        def tile_body(t, carry):
            for j in range(128 // LANES):
                sl = pl.ds(j * LANES, LANES)
                ls = [lv[t, e, sl] for e in range(E)]

                m1 = ls[0]
                i1 = zero
                for e in range(1, E):
                    gt = ls[e] > m1
                    m1 = jnp.where(gt, ls[e], m1)
                    i1 = jnp.where(gt, e, i1)

                m2 = negv
                i2 = zero
                for e in range(E):
                    cand = jnp.where(i1 == e, neg, ls[e])
                    gt = cand > m2
                    m2 = jnp.where(gt, cand, m2)
                    i2 = jnp.where(gt, e, i2)

                t2 = jnp.exp(m2 - m1)
                g1 = 1.0 / (1.0 + t2)
                g2 = t2 * g1

                gv[t, 0, sl] = g1
                gv[t, 1, sl] = g2
                iv[t, 0, sl] = i1
                iv[t, 1, sl] = i2
            return carry

        lax.fori_loop(0, tpw, tile_body, 0)

        pltpu.sync_copy(gv, gates_hbm.at[pl.ds(base, tpw)])
        pltpu.sync_copy(iv, idx_hbm.at[pl.ds(base, tpw)])

    return router


def kernel(x, W):
    B, S, _ = x.shape
    n_tok = B * S
    n_tiles = n_tok // 128
    x_flat = x.reshape(n_tok, D)
    logits_t = _gate_logits_t(x_flat, W.T)
    gates_t, idx_t = _make_router(n_tiles)(logits_t)

    def unpack(img, k):
        return img.reshape(B, S // 128, k, 128).transpose(0, 1, 3, 2).reshape(B, S, k)

    return (unpack(gates_t, K), unpack(idx_t, K), unpack(logits_t, E))


# speedup vs baseline: 2.4008x; 1.0340x over previous
import functools

import jax
import jax.numpy as jnp
from jax import lax
from jax.experimental import pallas as pl
from jax.experimental.pallas import tpu as pltpu
from jax.experimental.pallas import tpu_sc as plsc

D = 768
E = 8
K = 2
BT = 2048

NC = 2
NS = 16
NW = NC * NS
LANES = 16


def _gate_body(x_ref, wt_ref, outt_ref):
    acc = jnp.dot(x_ref[...], wt_ref[...], preferred_element_type=jnp.float32)
    outt_ref[...] = jnp.transpose(acc.reshape(BT // 128, 128, E), (0, 2, 1))


def _gate_logits_t(x_flat, wt):
    n_tok = x_flat.shape[0]
    grid = (n_tok // BT,)
    return pl.pallas_call(
        _gate_body,
        grid=grid,
        in_specs=[
            pl.BlockSpec((BT, D), lambda i: (i, 0)),
            pl.BlockSpec((D, E), lambda i: (0, 0)),
        ],
        out_specs=pl.BlockSpec((BT // 128, E, 128), lambda i: (i, 0, 0)),
        out_shape=jax.ShapeDtypeStruct((n_tok // 128, E, 128), jnp.float32),
    )(x_flat, wt)


def _make_router(n_tiles):
    tpw = n_tiles // NW

    mesh = plsc.VectorSubcoreMesh(core_axis_name="c", subcore_axis_name="s")

    @functools.partial(
        pl.kernel,
        mesh=mesh,
        compiler_params=pltpu.CompilerParams(
            needs_layout_passes=False, skip_device_barrier=True),
        out_type=[
            jax.ShapeDtypeStruct((n_tiles, K, 128), jnp.float32),
            jax.ShapeDtypeStruct((n_tiles, K, 128), jnp.int32),
        ],
        scratch_types=[
            pltpu.VMEM((tpw, E, 128), jnp.float32),
            pltpu.VMEM((tpw, K, 128), jnp.float32),
            pltpu.VMEM((tpw, K, 128), jnp.int32),
        ],
    )
    def router(lt_hbm, gates_hbm, idx_hbm, lv, gv, iv):
        wid = lax.axis_index("s") * NC + lax.axis_index("c")
        base = wid * tpw
        pltpu.sync_copy(lt_hbm.at[pl.ds(base, tpw)], lv)

        zero = jnp.zeros((LANES,), jnp.int32)
        neg = jnp.float32(-1e30)
        negv = jnp.full((LANES,), neg, jnp.float32)

        def tile_body(t, carry):
            for j in range(128 // LANES):
                sl = pl.ds(j * LANES, LANES)
                ls = [lv[t, e, sl] for e in range(E)]

                m1 = ls[0]
                i1 = zero
                for e in range(1, E):
                    gt = ls[e] > m1
                    m1 = jnp.where(gt, ls[e], m1)
                    i1 = jnp.where(gt, e, i1)

                m2 = negv
                i2 = zero
                for e in range(E):
                    cand = jnp.where(i1 == e, neg, ls[e])
                    gt = cand > m2
                    m2 = jnp.where(gt, cand, m2)
                    i2 = jnp.where(gt, e, i2)

                t2 = jnp.exp(m2 - m1)
                g1 = 1.0 / (1.0 + t2)
                g2 = t2 * g1

                gv[t, 0, sl] = g1
                gv[t, 1, sl] = g2
                iv[t, 0, sl] = i1
                iv[t, 1, sl] = i2
            return carry

        lax.fori_loop(0, tpw, tile_body, 0)

        pltpu.sync_copy(gv, gates_hbm.at[pl.ds(base, tpw)])
        pltpu.sync_copy(iv, idx_hbm.at[pl.ds(base, tpw)])

    return router


def kernel(x, W):
    B, S, _ = x.shape
    n_tok = B * S
    n_tiles = n_tok // 128
    x_flat = x.reshape(n_tok, D)
    logits_t = _gate_logits_t(x_flat, W.T)
    gates_t, idx_t = _make_router(n_tiles)(logits_t)

    def unpack(img, k):
        return img.reshape(B, S // 128, k, 128).transpose(0, 1, 3, 2).reshape(B, S, k)

    return (unpack(gates_t, K), unpack(idx_t, K), unpack(logits_t, E))


# speedup vs baseline: 2.4149x; 1.0058x over previous
import functools

import jax
import jax.numpy as jnp
from jax import lax
from jax.experimental import pallas as pl
from jax.experimental.pallas import tpu as pltpu
from jax.experimental.pallas import tpu_sc as plsc

D = 768
E = 8
K = 2
BT = 4096

NC = 2
NS = 16
NW = NC * NS
LANES = 16


def _gate_body(x_ref, wt_ref, outt_ref):
    acc = jnp.dot(x_ref[...], wt_ref[...], preferred_element_type=jnp.float32)
    outt_ref[...] = jnp.transpose(acc.reshape(BT // 128, 128, E), (0, 2, 1))


def _gate_logits_t(x_flat, wt):
    n_tok = x_flat.shape[0]
    grid = (n_tok // BT,)
    return pl.pallas_call(
        _gate_body,
        grid=grid,
        in_specs=[
            pl.BlockSpec((BT, D), lambda i: (i, 0)),
            pl.BlockSpec((D, E), lambda i: (0, 0)),
        ],
        out_specs=pl.BlockSpec((BT // 128, E, 128), lambda i: (i, 0, 0)),
        out_shape=jax.ShapeDtypeStruct((n_tok // 128, E, 128), jnp.float32),
    )(x_flat, wt)


def _make_router(n_tiles):
    tpw = n_tiles // NW

    mesh = plsc.VectorSubcoreMesh(core_axis_name="c", subcore_axis_name="s")

    @functools.partial(
        pl.kernel,
        mesh=mesh,
        compiler_params=pltpu.CompilerParams(
            needs_layout_passes=False, skip_device_barrier=True),
        out_type=[
            jax.ShapeDtypeStruct((n_tiles, K, 128), jnp.float32),
            jax.ShapeDtypeStruct((n_tiles, K, 128), jnp.int32),
        ],
        scratch_types=[
            pltpu.VMEM((tpw, E, 128), jnp.float32),
            pltpu.VMEM((tpw, K, 128), jnp.float32),
            pltpu.VMEM((tpw, K, 128), jnp.int32),
        ],
    )
    def router(lt_hbm, gates_hbm, idx_hbm, lv, gv, iv):
        wid = lax.axis_index("s") * NC + lax.axis_index("c")
        base = wid * tpw
        pltpu.sync_copy(lt_hbm.at[pl.ds(base, tpw)], lv)

        zero = jnp.zeros((LANES,), jnp.int32)
        neg = jnp.float32(-1e30)
        negv = jnp.full((LANES,), neg, jnp.float32)

        def tile_body(t, carry):
            for j in range(128 // LANES):
                sl = pl.ds(j * LANES, LANES)
                ls = [lv[t, e, sl] for e in range(E)]

                m1 = ls[0]
                i1 = zero
                for e in range(1, E):
                    gt = ls[e] > m1
                    m1 = jnp.where(gt, ls[e], m1)
                    i1 = jnp.where(gt, e, i1)

                m2 = negv
                i2 = zero
                for e in range(E):
                    cand = jnp.where(i1 == e, neg, ls[e])
                    gt = cand > m2
                    m2 = jnp.where(gt, cand, m2)
                    i2 = jnp.where(gt, e, i2)

                t2 = jnp.exp(m2 - m1)
                g1 = 1.0 / (1.0 + t2)
                g2 = t2 * g1

                gv[t, 0, sl] = g1
                gv[t, 1, sl] = g2
                iv[t, 0, sl] = i1
                iv[t, 1, sl] = i2
            return carry

        lax.fori_loop(0, tpw, tile_body, 0)

        pltpu.sync_copy(gv, gates_hbm.at[pl.ds(base, tpw)])
        pltpu.sync_copy(iv, idx_hbm.at[pl.ds(base, tpw)])

    return router


def kernel(x, W):
    B, S, _ = x.shape
    n_tok = B * S
    n_tiles = n_tok // 128
    x_flat = x.reshape(n_tok, D)
    logits_t = _gate_logits_t(x_flat, W.T)
    gates_t, idx_t = _make_router(n_tiles)(logits_t)

    def unpack(img, k):
        return img.reshape(B, S // 128, k, 128).transpose(0, 1, 3, 2).reshape(B, S, k)

    return (unpack(gates_t, K), unpack(idx_t, K), unpack(logits_t, E))
